# Initial kernel scaffold; baseline (speedup 1.0000x reference)
import jax
import jax.numpy as jnp
from jax.experimental import pallas as pl


def kernel(user_entity_table, relation_table, trans_matrix, heads, relations, positive_tails, negative_tails):
    raise NotImplementedError("write your pallas kernel here")


# speedup vs baseline: 1.7940x; 1.7940x over previous
import functools

import jax
import jax.numpy as jnp
from jax import lax
from jax.experimental import pallas as pl
from jax.experimental.pallas import tpu as pltpu
from jax.experimental.pallas import tpu_sc as plsc

_USER_NUM = 10000
_ENTITY_NUM = 100000
_RELATION_NUM = 40
_DIM = 128
_B = 16384
_REG = 1e-05

_CH = 128


def _make_gather(n_rows):
    info = plsc.get_sparse_core_info()
    nc, ns = info.num_cores, info.num_subcores
    nw = nc * ns
    per_w = n_rows // nw
    n_ch = per_w // _CH
    mesh = plsc.VectorSubcoreMesh(core_axis_name="c", subcore_axis_name="s")

    @functools.partial(
        pl.kernel,
        mesh=mesh,
        out_type=jax.ShapeDtypeStruct((n_rows, _DIM), jnp.float32),
        scratch_types=[
            pltpu.VMEM((_CH,), jnp.int32),
            pltpu.VMEM((_CH, _DIM), jnp.float32),
            pltpu.SemaphoreType.DMA,
        ],
    )
    def gather_k(table_hbm, idx_hbm, out_hbm, idx_v, rows_v, sem):
        wid = lax.axis_index("s") * nc + lax.axis_index("c")
        base = wid * per_w

        def body(j, carry):
            off = base + j * _CH
            pltpu.sync_copy(idx_hbm.at[pl.ds(off, _CH)], idx_v)
            pltpu.async_copy(table_hbm.at[idx_v], rows_v, sem).wait()
            pltpu.sync_copy(rows_v, out_hbm.at[pl.ds(off, _CH)])
            return carry

        lax.fori_loop(0, n_ch, body, 0)

    return gather_k


_BK = 512
_NB = _B // _BK


def _tc_body(rel_ref, hb_ref, pb_ref, nb_ref, rtab_ref, trans_ref, out_ref,
             s_ref, acc_ref):
    i = pl.program_id(0)
    rel = rel_ref[0, 0, :]
    rel3 = jnp.concatenate([rel, rel, rel], axis=0)

    s_ref[0:_BK, :] = hb_ref[...]
    s_ref[_BK:2 * _BK, :] = pb_ref[...]
    s_ref[2 * _BK:3 * _BK, :] = nb_ref[...]
    acc_ref[...] = jnp.zeros((3 * _BK, _DIM), jnp.float32)

    def body(r, carry):
        m = (rel3 == r).astype(jnp.float32)[:, None]
        w_r = trans_ref[r, :, :]
        acc_ref[...] += jnp.dot(s_ref[...] * m, w_r,
                                preferred_element_type=jnp.float32)
        return carry

    lax.fori_loop(0, _RELATION_NUM, body, 0)

    th = acc_ref[0:_BK, :]
    tp = acc_ref[_BK:2 * _BK, :]
    tn = acc_ref[2 * _BK:3 * _BK, :]

    oh = (rel[:, None] == lax.broadcasted_iota(jnp.int32, (1, _RELATION_NUM), 1)
          ).astype(jnp.float32)
    remb = jnp.dot(oh, rtab_ref[...], preferred_element_type=jnp.float32)

    pos = jnp.sum(jnp.square(th + remb - tp), axis=1)
    neg = jnp.sum(jnp.square(th + remb - tn), axis=1)
    d = neg - pos
    ls = jnp.minimum(d, 0.0) - jnp.log1p(jnp.exp(-jnp.abs(d)))

    rows = lax.broadcasted_iota(jnp.int32, (8, _DIM), 0)
    cols = lax.broadcasted_iota(jnp.int32, (8, _DIM), 1)
    partial = (jnp.sum(ls) * (rows == 0) + jnp.sum(th * th) * (rows == 1)
               + jnp.sum(remb * remb) * (rows == 2)
               + jnp.sum(tp * tp) * (rows == 3)
               + jnp.sum(tn * tn) * (rows == 4)).astype(jnp.float32)

    @pl.when(i == 0)
    def _():
        out_ref[...] = jnp.zeros((8, _DIM), jnp.float32)

    out_ref[...] += partial

    @pl.when(i == _NB - 1)
    def _():
        vals = out_ref[...]
        inv = 1.0 / _DIM

        def tot(r):
            return jnp.sum(vals * (rows == r).astype(jnp.float32)) * inv

        kg = -tot(0) / _B
        l2 = (tot(1) + tot(2) + tot(3) + tot(4)) / (2.0 * _B)
        res = kg + _REG * l2
        out_ref[...] = res * ((rows == 0) & (cols == 0)).astype(jnp.float32)


def _tc_call(rows3, rel3d, rtab, trans):
    return pl.pallas_call(
        _tc_body,
        grid=(_NB,),
        in_specs=[
            pl.BlockSpec((1, 1, _BK), lambda i: (i, 0, 0)),
            pl.BlockSpec((_BK, _DIM), lambda i: (i, 0)),
            pl.BlockSpec((_BK, _DIM), lambda i: (i + _NB, 0)),
            pl.BlockSpec((_BK, _DIM), lambda i: (i + 2 * _NB, 0)),
            pl.BlockSpec((_RELATION_NUM, _DIM), lambda i: (0, 0)),
            pl.BlockSpec((_RELATION_NUM, _DIM, _DIM), lambda i: (0, 0, 0)),
        ],
        out_specs=pl.BlockSpec((8, _DIM), lambda i: (0, 0)),
        out_shape=jax.ShapeDtypeStruct((8, _DIM), jnp.float32),
        scratch_shapes=[
            pltpu.VMEM((3 * _BK, _DIM), jnp.float32),
            pltpu.VMEM((3 * _BK, _DIM), jnp.float32),
        ],
    )(rel3d, rows3, rows3, rows3, rtab, trans)


def kernel(user_entity_table, relation_table, trans_matrix, heads, relations,
           positive_tails, negative_tails):
    idx = jnp.concatenate([
        heads.astype(jnp.int32),
        positive_tails.astype(jnp.int32),
        negative_tails.astype(jnp.int32),
    ])
    rows3 = _make_gather(3 * _B)(user_entity_table, idx)
    rel3d = relations.astype(jnp.int32).reshape(_NB, 1, _BK)
    out = _tc_call(rows3, rel3d, relation_table, trans_matrix)
    return out[0, 0]


# speedup vs baseline: 5.3915x; 3.0053x over previous
import functools

import jax
import jax.numpy as jnp
from jax import lax
from jax.experimental import pallas as pl
from jax.experimental.pallas import tpu as pltpu
from jax.experimental.pallas import tpu_sc as plsc

_RELATION_NUM = 40
_DIM = 128
_B = 16384
_REG = 1e-05

_NSUB = 16
_NW = 32
_PW = _B // _NW
_NV = _PW // 16
_HALF = _B // 2
_CH = 128


def _make_sort_gather():
    mesh = plsc.VectorSubcoreMesh(core_axis_name="c", subcore_axis_name="s")

    @functools.partial(
        pl.kernel,
        mesh=mesh,
        out_type=(
            jax.ShapeDtypeStruct((_B, _DIM), jnp.float32),
            jax.ShapeDtypeStruct((_B, _DIM), jnp.float32),
            jax.ShapeDtypeStruct((_B, _DIM), jnp.float32),
            jax.ShapeDtypeStruct((_B,), jnp.int32),
        ),
        scratch_types=[
            pltpu.VMEM((_PW,), jnp.int32),
            pltpu.VMEM((_PW,), jnp.int32),
            pltpu.VMEM((_PW,), jnp.int32),
            pltpu.VMEM((_PW,), jnp.int32),
            pltpu.VMEM((_PW + 16,), jnp.int32),
            pltpu.VMEM((48,), jnp.int32),
            pltpu.VMEM((48,), jnp.int32),
            pltpu.VMEM((4, _CH), jnp.int32),
            pltpu.VMEM((_PW,), jnp.int32),
            pltpu.VMEM((_PW,), jnp.int32),
            pltpu.VMEM((_PW,), jnp.int32),
            pltpu.VMEM((_PW,), jnp.int32),
            pltpu.VMEM((_NSUB * 48,), jnp.int32),
            pltpu.VMEM((_CH, _DIM), jnp.float32),
            pltpu.VMEM_SHARED((_NSUB * 48,), jnp.int32),
            pltpu.SemaphoreType.DMA,
        ],
        compiler_params=pltpu.CompilerParams(needs_layout_passes=False),
    )
    def sg(table, heads, ptails, ntails, rels,
           out_h, out_p, out_n, out_r,
           relv, headv, ptv, ntv, ordbuf, cnt48, delta48, pos2d,
           hsort, psort, nsort, rsort, hall, rows_v, hists_sh, sem):
        c = lax.axis_index("c")
        s = lax.axis_index("s")
        gbase = c * _HALF + s * _PW
        pltpu.sync_copy(rels.at[pl.ds(gbase, _PW)], relv)
        pltpu.sync_copy(heads.at[pl.ds(gbase, _PW)], headv)
        pltpu.sync_copy(ptails.at[pl.ds(gbase, _PW)], ptv)
        pltpu.sync_copy(ntails.at[pl.ds(gbase, _PW)], ntv)

        lane = lax.broadcasted_iota(jnp.int32, (16,), 0)
        zero = jnp.zeros((16,), jnp.int32)

        def bc(x):
            return jnp.broadcast_to(x, (16,))

        def over_r(r, carry):
            off0, cnt0, cnt1, cnt2, lst0, lst1, lst2 = carry
            offv = bc(off0)
            lst0 = jnp.where(lane == bc(r), offv, lst0)
            lst1 = jnp.where(lane == bc(r - 16), offv, lst1)
            lst2 = jnp.where(lane == bc(r - 32), offv, lst2)

            def over_v(j, off):
                v = relv[pl.ds(j * 16, 16)]
                m = v == bc(r)
                mi = m.astype(jnp.int32)
                excl = plsc.cumsum(mi) - mi
                plsc.store_scatter(ordbuf, [bc(off) + excl],
                                   lane + bc(j * 16), mask=m)
                return off + jnp.sum(mi)

            off1 = lax.fori_loop(0, _NV, over_v, off0)
            crv = bc(off1 - off0)
            cnt0 = jnp.where(lane == bc(r), crv, cnt0)
            cnt1 = jnp.where(lane == bc(r - 16), crv, cnt1)
            cnt2 = jnp.where(lane == bc(r - 32), crv, cnt2)
            return off1, cnt0, cnt1, cnt2, lst0, lst1, lst2

        init = (0, zero, zero, zero, zero, zero, zero)
        _, cnt0, cnt1, cnt2, lst0, lst1, lst2 = lax.fori_loop(
            0, _RELATION_NUM, over_r, init)
        cnt48[pl.ds(0, 16)] = cnt0
        cnt48[pl.ds(16, 16)] = cnt1
        cnt48[pl.ds(32, 16)] = cnt2

        pltpu.sync_copy(cnt48, hists_sh.at[pl.ds(s * 48, 48)])
        plsc.subcore_barrier()
        pltpu.sync_copy(hists_sh, hall)

        t0 = t1 = t2 = p0 = p1 = p2 = zero
        for w in range(_NSUB):
            h0 = hall[pl.ds(w * 48, 16)]
            h1 = hall[pl.ds(w * 48 + 16, 16)]
            h2 = hall[pl.ds(w * 48 + 32, 16)]
            use = bc((w < s).astype(jnp.int32))
            t0 += h0
            t1 += h1
            t2 += h2
            p0 += h0 * use
            p1 += h1 * use
            p2 += h2 * use
        s0 = bc(jnp.sum(t0))
        s1 = bc(jnp.sum(t1))
        e0 = plsc.cumsum(t0) - t0
        e1 = plsc.cumsum(t1) - t1 + s0
        e2 = plsc.cumsum(t2) - t2 + s0 + s1
        basev = bc(c * _HALF)
        delta48[pl.ds(0, 16)] = e0 + p0 + basev - lst0
        delta48[pl.ds(16, 16)] = e1 + p1 + basev - lst1
        delta48[pl.ds(32, 16)] = e2 + p2 + basev - lst2

        for j in range(_NV):
            ordv = ordbuf[pl.ds(j * 16, 16)]
            rsv = plsc.load_gather(relv, [ordv])
            dv = plsc.load_gather(delta48, [rsv])
            pos2d[j // 8, pl.ds((j % 8) * 16, 16)] = dv + lane + bc(j * 16)
            hsort[pl.ds(j * 16, 16)] = plsc.load_gather(headv, [ordv])
            psort[pl.ds(j * 16, 16)] = plsc.load_gather(ptv, [ordv])
            nsort[pl.ds(j * 16, 16)] = plsc.load_gather(ntv, [ordv])
            rsort[pl.ds(j * 16, 16)] = rsv

        for ch in range(_PW // _CH):
            sl = pl.ds(ch * _CH, _CH)
            pltpu.async_copy(table.at[hsort.at[sl]], rows_v, sem).wait()
            pltpu.async_copy(rows_v, out_h.at[pos2d.at[ch]], sem).wait()
            pltpu.async_copy(table.at[psort.at[sl]], rows_v, sem).wait()
            pltpu.async_copy(rows_v, out_p.at[pos2d.at[ch]], sem).wait()
            pltpu.async_copy(table.at[nsort.at[sl]], rows_v, sem).wait()
            pltpu.async_copy(rows_v, out_n.at[pos2d.at[ch]], sem).wait()
            pltpu.async_copy(rsort.at[sl], out_r.at[pos2d.at[ch]], sem).wait()

    return sg


_BK = 512
_NB = _B // _BK


def _tc_body(rel_ref, hb_ref, pb_ref, nb_ref, rtab_ref, trans_ref, out_ref,
             s_ref, acc_ref):
    i = pl.program_id(0)
    rel = rel_ref[0, 0, :]
    rel3 = jnp.concatenate([rel, rel, rel], axis=0)
    lo = jnp.min(rel)
    hi = jnp.max(rel)

    s_ref[0:_BK, :] = hb_ref[...]
    s_ref[_BK:2 * _BK, :] = pb_ref[...]
    s_ref[2 * _BK:3 * _BK, :] = nb_ref[...]
    acc_ref[...] = jnp.zeros((3 * _BK, _DIM), jnp.float32)

    def body(r, carry):
        m = (rel3 == r).astype(jnp.float32)[:, None]
        w_r = trans_ref[r, :, :]
        acc_ref[...] += jnp.dot(s_ref[...] * m, w_r,
                                preferred_element_type=jnp.float32)
        return carry

    lax.fori_loop(lo, hi + 1, body, 0)

    th = acc_ref[0:_BK, :]
    tp = acc_ref[_BK:2 * _BK, :]
    tn = acc_ref[2 * _BK:3 * _BK, :]

    oh = (rel[:, None] == lax.broadcasted_iota(jnp.int32, (1, _RELATION_NUM), 1)
          ).astype(jnp.float32)
    remb = jnp.dot(oh, rtab_ref[...], preferred_element_type=jnp.float32)

    pos = jnp.sum(jnp.square(th + remb - tp), axis=1)
    neg = jnp.sum(jnp.square(th + remb - tn), axis=1)
    d = neg - pos
    ls = jnp.minimum(d, 0.0) - jnp.log1p(jnp.exp(-jnp.abs(d)))

    rows = lax.broadcasted_iota(jnp.int32, (8, _DIM), 0)
    cols = lax.broadcasted_iota(jnp.int32, (8, _DIM), 1)
    partial = (jnp.sum(ls) * (rows == 0) + jnp.sum(th * th) * (rows == 1)
               + jnp.sum(remb * remb) * (rows == 2)
               + jnp.sum(tp * tp) * (rows == 3)
               + jnp.sum(tn * tn) * (rows == 4)).astype(jnp.float32)

    @pl.when(i == 0)
    def _():
        out_ref[...] = jnp.zeros((8, _DIM), jnp.float32)

    out_ref[...] += partial

    @pl.when(i == _NB - 1)
    def _():
        vals = out_ref[...]
        inv = 1.0 / _DIM

        def tot(r):
            return jnp.sum(vals * (rows == r).astype(jnp.float32)) * inv

        kg = -tot(0) / _B
        l2 = (tot(1) + tot(2) + tot(3) + tot(4)) / (2.0 * _B)
        res = kg + _REG * l2
        out_ref[...] = res * ((rows == 0) & (cols == 0)).astype(jnp.float32)


def _tc_call(rows_h, rows_p, rows_n, rel3d, rtab, trans):
    return pl.pallas_call(
        _tc_body,
        grid=(_NB,),
        in_specs=[
            pl.BlockSpec((1, 1, _BK), lambda i: (i, 0, 0)),
            pl.BlockSpec((_BK, _DIM), lambda i: (i, 0)),
            pl.BlockSpec((_BK, _DIM), lambda i: (i, 0)),
            pl.BlockSpec((_BK, _DIM), lambda i: (i, 0)),
            pl.BlockSpec((_RELATION_NUM, _DIM), lambda i: (0, 0)),
            pl.BlockSpec((_RELATION_NUM, _DIM, _DIM), lambda i: (0, 0, 0)),
        ],
        out_specs=pl.BlockSpec((8, _DIM), lambda i: (0, 0)),
        out_shape=jax.ShapeDtypeStruct((8, _DIM), jnp.float32),
        scratch_shapes=[
            pltpu.VMEM((3 * _BK, _DIM), jnp.float32),
            pltpu.VMEM((3 * _BK, _DIM), jnp.float32),
        ],
    )(rel3d, rows_h, rows_p, rows_n, rtab, trans)


def kernel(user_entity_table, relation_table, trans_matrix, heads, relations,
           positive_tails, negative_tails):
    rows_h, rows_p, rows_n, rel_s = _make_sort_gather()(
        user_entity_table,
        heads.astype(jnp.int32),
        positive_tails.astype(jnp.int32),
        negative_tails.astype(jnp.int32),
        relations.astype(jnp.int32),
    )
    rel3d = rel_s.reshape(_NB, 1, _BK)
    out = _tc_call(rows_h, rows_p, rows_n, rel3d, relation_table, trans_matrix)
    return out[0, 0]


# speedup vs baseline: 5.8410x; 1.0834x over previous
import functools

import jax
import jax.numpy as jnp
from jax import lax
from jax.experimental import pallas as pl
from jax.experimental.pallas import tpu as pltpu
from jax.experimental.pallas import tpu_sc as plsc

_RELATION_NUM = 40
_DIM = 128
_B = 16384
_REG = 1e-05

_NSUB = 16
_NW = 32
_PW = _B // _NW
_NV = _PW // 16
_HALF = _B // 2
_CH = 128


def _make_sort_gather():
    mesh = plsc.VectorSubcoreMesh(core_axis_name="c", subcore_axis_name="s")

    @functools.partial(
        pl.kernel,
        mesh=mesh,
        out_type=(
            jax.ShapeDtypeStruct((_B, _DIM), jnp.float32),
            jax.ShapeDtypeStruct((_B, _DIM), jnp.float32),
            jax.ShapeDtypeStruct((_B, _DIM), jnp.float32),
            jax.ShapeDtypeStruct((_B,), jnp.int32),
        ),
        scratch_types=[
            pltpu.VMEM((_PW,), jnp.int32),
            pltpu.VMEM((_PW,), jnp.int32),
            pltpu.VMEM((_PW,), jnp.int32),
            pltpu.VMEM((_PW,), jnp.int32),
            pltpu.VMEM((_PW + 16,), jnp.int32),
            pltpu.VMEM((48,), jnp.int32),
            pltpu.VMEM((48,), jnp.int32),
            pltpu.VMEM((4, _CH), jnp.int32),
            pltpu.VMEM((_PW,), jnp.int32),
            pltpu.VMEM((_PW,), jnp.int32),
            pltpu.VMEM((_PW,), jnp.int32),
            pltpu.VMEM((_PW,), jnp.int32),
            pltpu.VMEM((_NSUB * 48,), jnp.int32),
            [pltpu.VMEM((_CH, _DIM), jnp.float32) for _ in range(6)],
            pltpu.VMEM_SHARED((_NSUB * 48,), jnp.int32),
            [pltpu.SemaphoreType.DMA for _ in range(6)],
            pltpu.SemaphoreType.DMA,
        ],
        compiler_params=pltpu.CompilerParams(needs_layout_passes=False),
    )
    def sg(table, heads, ptails, ntails, rels,
           out_h, out_p, out_n, out_r,
           relv, headv, ptv, ntv, ordbuf, cnt48, delta48, pos2d,
           hsort, psort, nsort, rsort, hall, bufs, hists_sh, sems, lsem):
        c = lax.axis_index("c")
        s = lax.axis_index("s")
        gbase = c * _HALF + s * _PW
        ld_r = pltpu.async_copy(rels.at[pl.ds(gbase, _PW)], relv, lsem)
        ld_h = pltpu.async_copy(heads.at[pl.ds(gbase, _PW)], headv, sems[0])
        ld_p = pltpu.async_copy(ptails.at[pl.ds(gbase, _PW)], ptv, sems[1])
        ld_n = pltpu.async_copy(ntails.at[pl.ds(gbase, _PW)], ntv, sems[2])
        ld_r.wait()

        lane = lax.broadcasted_iota(jnp.int32, (16,), 0)
        zero = jnp.zeros((16,), jnp.int32)

        def bc(x):
            return jnp.broadcast_to(x, (16,))

        def over_r(r, carry):
            off0, cnt0, cnt1, cnt2, lst0, lst1, lst2 = carry
            offv = bc(off0)
            lst0 = jnp.where(lane == bc(r), offv, lst0)
            lst1 = jnp.where(lane == bc(r - 16), offv, lst1)
            lst2 = jnp.where(lane == bc(r - 32), offv, lst2)

            def over_v(j, off):
                v = relv[pl.ds(j * 16, 16)]
                m = v == bc(r)
                mi = m.astype(jnp.int32)
                excl = plsc.cumsum(mi) - mi
                plsc.store_scatter(ordbuf, [bc(off) + excl],
                                   lane + bc(j * 16), mask=m)
                return off + jnp.sum(mi)

            off1 = lax.fori_loop(0, _NV, over_v, off0)
            crv = bc(off1 - off0)
            cnt0 = jnp.where(lane == bc(r), crv, cnt0)
            cnt1 = jnp.where(lane == bc(r - 16), crv, cnt1)
            cnt2 = jnp.where(lane == bc(r - 32), crv, cnt2)
            return off1, cnt0, cnt1, cnt2, lst0, lst1, lst2

        init = (0, zero, zero, zero, zero, zero, zero)
        _, cnt0, cnt1, cnt2, lst0, lst1, lst2 = lax.fori_loop(
            0, _RELATION_NUM, over_r, init)
        cnt48[pl.ds(0, 16)] = cnt0
        cnt48[pl.ds(16, 16)] = cnt1
        cnt48[pl.ds(32, 16)] = cnt2

        pltpu.sync_copy(cnt48, hists_sh.at[pl.ds(s * 48, 48)])
        plsc.subcore_barrier()
        pltpu.sync_copy(hists_sh, hall)

        t0 = t1 = t2 = p0 = p1 = p2 = zero
        for w in range(_NSUB):
            h0 = hall[pl.ds(w * 48, 16)]
            h1 = hall[pl.ds(w * 48 + 16, 16)]
            h2 = hall[pl.ds(w * 48 + 32, 16)]
            use = bc((w < s).astype(jnp.int32))
            t0 += h0
            t1 += h1
            t2 += h2
            p0 += h0 * use
            p1 += h1 * use
            p2 += h2 * use
        s0 = bc(jnp.sum(t0))
        s1 = bc(jnp.sum(t1))
        e0 = plsc.cumsum(t0) - t0
        e1 = plsc.cumsum(t1) - t1 + s0
        e2 = plsc.cumsum(t2) - t2 + s0 + s1
        basev = bc(c * _HALF)
        delta48[pl.ds(0, 16)] = e0 + p0 + basev - lst0
        delta48[pl.ds(16, 16)] = e1 + p1 + basev - lst1
        delta48[pl.ds(32, 16)] = e2 + p2 + basev - lst2

        ld_h.wait()
        ld_p.wait()
        ld_n.wait()
        for j in range(_NV):
            ordv = ordbuf[pl.ds(j * 16, 16)]
            rsv = plsc.load_gather(relv, [ordv])
            dv = plsc.load_gather(delta48, [rsv])
            pos2d[j // 8, pl.ds((j % 8) * 16, 16)] = dv + lane + bc(j * 16)
            hsort[pl.ds(j * 16, 16)] = plsc.load_gather(headv, [ordv])
            psort[pl.ds(j * 16, 16)] = plsc.load_gather(ptv, [ordv])
            nsort[pl.ds(j * 16, 16)] = plsc.load_gather(ntv, [ordv])
            rsort[pl.ds(j * 16, 16)] = rsv

        rel_descs = [
            pltpu.async_copy(rsort.at[pl.ds(ch * _CH, _CH)],
                             out_r.at[pos2d.at[ch]], lsem)
            for ch in range(_PW // _CH)
        ]

        srcs = [hsort, psort, nsort]
        outs = [out_h, out_p, out_n]
        njobs = 3 * (_PW // _CH)

        def job(k):
            t, ch = k % 3, k // 3
            return srcs[t], outs[t], ch

        dg = {}
        dsc = {}
        for k in range(6):
            src, _, ch = job(k)
            dg[k] = pltpu.async_copy(table.at[src.at[pl.ds(ch * _CH, _CH)]],
                                     bufs[k], sems[k])
        for k in range(njobs):
            slot = k % 6
            dg[k].wait()
            _, out, ch = job(k)
            dsc[k] = pltpu.async_copy(bufs[slot], out.at[pos2d.at[ch]],
                                      sems[slot])
            if k + 6 < njobs:
                dsc[k].wait()
                src, _, ch2 = job(k + 6)
                dg[k + 6] = pltpu.async_copy(
                    table.at[src.at[pl.ds(ch2 * _CH, _CH)]],
                    bufs[slot], sems[slot])
        for k in range(njobs - 6, njobs):
            dsc[k].wait()
        for dsc_rel in rel_descs:
            dsc_rel.wait()

    return sg


_BK = 1024
_NB = _B // _BK


def _tc_body(rel_ref, hb_ref, pb_ref, nb_ref, rtab_ref, trans_ref, out_ref,
             s_ref, acc_ref):
    i = pl.program_id(0)
    rel = rel_ref[0, 0, :]
    rel3 = jnp.concatenate([rel, rel, rel], axis=0)
    lo = jnp.min(rel)
    hi = jnp.max(rel)

    s_ref[0:_BK, :] = hb_ref[...]
    s_ref[_BK:2 * _BK, :] = pb_ref[...]
    s_ref[2 * _BK:3 * _BK, :] = nb_ref[...]
    acc_ref[...] = jnp.zeros((3 * _BK, _DIM), jnp.float32)

    def body(r, carry):
        m = (rel3 == r).astype(jnp.float32)[:, None]
        w_r = trans_ref[r, :, :]
        acc_ref[...] += jnp.dot(s_ref[...] * m, w_r,
                                preferred_element_type=jnp.float32)
        return carry

    lax.fori_loop(lo, hi + 1, body, 0)

    th = acc_ref[0:_BK, :]
    tp = acc_ref[_BK:2 * _BK, :]
    tn = acc_ref[2 * _BK:3 * _BK, :]

    oh = (rel[:, None] == lax.broadcasted_iota(jnp.int32, (1, _RELATION_NUM), 1)
          ).astype(jnp.float32)
    remb = jnp.dot(oh, rtab_ref[...], preferred_element_type=jnp.float32)

    pos = jnp.sum(jnp.square(th + remb - tp), axis=1)
    neg = jnp.sum(jnp.square(th + remb - tn), axis=1)
    d = neg - pos
    ls = jnp.minimum(d, 0.0) - jnp.log1p(jnp.exp(-jnp.abs(d)))

    rows = lax.broadcasted_iota(jnp.int32, (8, _DIM), 0)
    cols = lax.broadcasted_iota(jnp.int32, (8, _DIM), 1)
    partial = (jnp.sum(ls) * (rows == 0) + jnp.sum(th * th) * (rows == 1)
               + jnp.sum(remb * remb) * (rows == 2)
               + jnp.sum(tp * tp) * (rows == 3)
               + jnp.sum(tn * tn) * (rows == 4)).astype(jnp.float32)

    @pl.when(i == 0)
    def _():
        out_ref[...] = jnp.zeros((8, _DIM), jnp.float32)

    out_ref[...] += partial

    @pl.when(i == _NB - 1)
    def _():
        vals = out_ref[...]
        inv = 1.0 / _DIM

        def tot(r):
            return jnp.sum(vals * (rows == r).astype(jnp.float32)) * inv

        kg = -tot(0) / _B
        l2 = (tot(1) + tot(2) + tot(3) + tot(4)) / (2.0 * _B)
        res = kg + _REG * l2
        out_ref[...] = res * ((rows == 0) & (cols == 0)).astype(jnp.float32)


def _tc_call(rows_h, rows_p, rows_n, rel3d, rtab, trans):
    return pl.pallas_call(
        _tc_body,
        grid=(_NB,),
        in_specs=[
            pl.BlockSpec((1, 1, _BK), lambda i: (i, 0, 0)),
            pl.BlockSpec((_BK, _DIM), lambda i: (i, 0)),
            pl.BlockSpec((_BK, _DIM), lambda i: (i, 0)),
            pl.BlockSpec((_BK, _DIM), lambda i: (i, 0)),
            pl.BlockSpec((_RELATION_NUM, _DIM), lambda i: (0, 0)),
            pl.BlockSpec((_RELATION_NUM, _DIM, _DIM), lambda i: (0, 0, 0)),
        ],
        out_specs=pl.BlockSpec((8, _DIM), lambda i: (0, 0)),
        out_shape=jax.ShapeDtypeStruct((8, _DIM), jnp.float32),
        scratch_shapes=[
            pltpu.VMEM((3 * _BK, _DIM), jnp.float32),
            pltpu.VMEM((3 * _BK, _DIM), jnp.float32),
        ],
    )(rel3d, rows_h, rows_p, rows_n, rtab, trans)


def kernel(user_entity_table, relation_table, trans_matrix, heads, relations,
           positive_tails, negative_tails):
    rows_h, rows_p, rows_n, rel_s = _make_sort_gather()(
        user_entity_table,
        heads.astype(jnp.int32),
        positive_tails.astype(jnp.int32),
        negative_tails.astype(jnp.int32),
        relations.astype(jnp.int32),
    )
    rel3d = rel_s.reshape(_NB, 1, _BK)
    out = _tc_call(rows_h, rows_p, rows_n, rel3d, relation_table, trans_matrix)
    return out[0, 0]


# speedup vs baseline: 6.3159x; 1.0813x over previous
import functools

import jax
import jax.numpy as jnp
from jax import lax
from jax.experimental import pallas as pl
from jax.experimental.pallas import tpu as pltpu
from jax.experimental.pallas import tpu_sc as plsc

_RELATION_NUM = 40
_DIM = 128
_B = 16384
_REG = 1e-05

_NSUB = 16
_NW = 32
_PW = _B // _NW
_NV = _PW // 16
_HALF = _B // 2
_CH = 128


def _make_sort_gather():
    mesh = plsc.VectorSubcoreMesh(core_axis_name="c", subcore_axis_name="s")

    @functools.partial(
        pl.kernel,
        mesh=mesh,
        out_type=(
            jax.ShapeDtypeStruct((_B, _DIM), jnp.float32),
            jax.ShapeDtypeStruct((_B, _DIM), jnp.float32),
            jax.ShapeDtypeStruct((_B, _DIM), jnp.float32),
            jax.ShapeDtypeStruct((_B,), jnp.int32),
        ),
        scratch_types=[
            pltpu.VMEM((_PW,), jnp.int32),
            pltpu.VMEM((_PW,), jnp.int32),
            pltpu.VMEM((_PW,), jnp.int32),
            pltpu.VMEM((_PW,), jnp.int32),
            pltpu.VMEM((_PW + 16,), jnp.int32),
            pltpu.VMEM((_PW,), jnp.int32),
            pltpu.VMEM((16,), jnp.int32),
            pltpu.VMEM((16,), jnp.int32),
            pltpu.VMEM((48,), jnp.int32),
            pltpu.VMEM((48,), jnp.int32),
            pltpu.VMEM((48,), jnp.int32),
            pltpu.VMEM((4, _CH), jnp.int32),
            pltpu.VMEM((_PW,), jnp.int32),
            pltpu.VMEM((_PW,), jnp.int32),
            pltpu.VMEM((_PW,), jnp.int32),
            pltpu.VMEM((_PW,), jnp.int32),
            pltpu.VMEM((_NSUB * 48,), jnp.int32),
            [pltpu.VMEM((_CH, _DIM), jnp.float32) for _ in range(6)],
            pltpu.VMEM_SHARED((_NSUB * 48,), jnp.int32),
            [pltpu.SemaphoreType.DMA for _ in range(6)],
            pltpu.SemaphoreType.DMA,
        ],
        compiler_params=pltpu.CompilerParams(needs_layout_passes=False),
    )
    def sg(table, heads, ptails, ntails, rels,
           out_h, out_p, out_n, out_r,
           relv, headv, ptv, ntv, ordbuf, rankbuf, tmp16, chg16, cnt48,
           lst48, delta48, pos2d,
           hsort, psort, nsort, rsort, hall, bufs, hists_sh, sems, lsem):
        c = lax.axis_index("c")
        s = lax.axis_index("s")
        gbase = c * _HALF + s * _PW
        ld_r = pltpu.async_copy(rels.at[pl.ds(gbase, _PW)], relv, lsem)
        ld_h = pltpu.async_copy(heads.at[pl.ds(gbase, _PW)], headv, sems[0])
        ld_p = pltpu.async_copy(ptails.at[pl.ds(gbase, _PW)], ptv, sems[1])
        ld_n = pltpu.async_copy(ntails.at[pl.ds(gbase, _PW)], ntv, sems[2])
        ld_r.wait()

        lane = lax.broadcasted_iota(jnp.int32, (16,), 0)
        zero = jnp.zeros((16,), jnp.int32)

        def bc(x):
            return jnp.broadcast_to(x, (16,))

        cnt48[pl.ds(0, 16)] = zero
        cnt48[pl.ds(16, 16)] = zero
        cnt48[pl.ds(32, 16)] = zero

        def pass1(j, carry):
            v = relv[pl.ds(j * 16, 16)]
            ks, vs = plsc.sort_key_val(v, lane)
            tmp16[...] = ks
            prev = plsc.load_gather(tmp16, [jnp.maximum(lane - 1, 0)])
            chg = ((ks != prev) | (lane == 0)).astype(jnp.int32)
            chg16[...] = chg
            nxt = plsc.load_gather(chg16, [jnp.minimum(lane + 1, 15)])
            lastchg = plsc.cummax(lane * chg)
            runrank = lane - lastchg
            base = plsc.load_gather(cnt48, [ks])
            localrank = base + runrank
            endm = (nxt == 1) | (lane == 15)
            plsc.store_scatter(cnt48, [ks], localrank + 1, mask=endm)
            plsc.store_scatter(rankbuf, [bc(j * 16) + vs], localrank)
            return carry

        lax.fori_loop(0, _NV, pass1, 0)

        cnt0 = cnt48[pl.ds(0, 16)]
        cnt1 = cnt48[pl.ds(16, 16)]
        cnt2 = cnt48[pl.ds(32, 16)]
        cs0 = bc(jnp.sum(cnt0))
        cs1 = bc(jnp.sum(cnt1))
        lst0 = plsc.cumsum(cnt0) - cnt0
        lst1 = plsc.cumsum(cnt1) - cnt1 + cs0
        lst2 = plsc.cumsum(cnt2) - cnt2 + cs0 + cs1
        lst48[pl.ds(0, 16)] = lst0
        lst48[pl.ds(16, 16)] = lst1
        lst48[pl.ds(32, 16)] = lst2

        def pass2(j, carry):
            v = relv[pl.ds(j * 16, 16)]
            rk = rankbuf[pl.ds(j * 16, 16)]
            basel = plsc.load_gather(lst48, [v])
            plsc.store_scatter(ordbuf, [basel + rk], lane + bc(j * 16))
            return carry

        lax.fori_loop(0, _NV, pass2, 0)

        pltpu.sync_copy(cnt48, hists_sh.at[pl.ds(s * 48, 48)])
        plsc.subcore_barrier()
        pltpu.sync_copy(hists_sh, hall)

        t0 = t1 = t2 = p0 = p1 = p2 = zero
        for w in range(_NSUB):
            h0 = hall[pl.ds(w * 48, 16)]
            h1 = hall[pl.ds(w * 48 + 16, 16)]
            h2 = hall[pl.ds(w * 48 + 32, 16)]
            use = bc((w < s).astype(jnp.int32))
            t0 += h0
            t1 += h1
            t2 += h2
            p0 += h0 * use
            p1 += h1 * use
            p2 += h2 * use
        s0 = bc(jnp.sum(t0))
        s1 = bc(jnp.sum(t1))
        e0 = plsc.cumsum(t0) - t0
        e1 = plsc.cumsum(t1) - t1 + s0
        e2 = plsc.cumsum(t2) - t2 + s0 + s1
        basev = bc(c * _HALF)
        delta48[pl.ds(0, 16)] = e0 + p0 + basev - lst0
        delta48[pl.ds(16, 16)] = e1 + p1 + basev - lst1
        delta48[pl.ds(32, 16)] = e2 + p2 + basev - lst2

        ld_h.wait()
        ld_p.wait()
        ld_n.wait()
        for j in range(_NV):
            ordv = ordbuf[pl.ds(j * 16, 16)]
            rsv = plsc.load_gather(relv, [ordv])
            dv = plsc.load_gather(delta48, [rsv])
            pos2d[j // 8, pl.ds((j % 8) * 16, 16)] = dv + lane + bc(j * 16)
            hsort[pl.ds(j * 16, 16)] = plsc.load_gather(headv, [ordv])
            psort[pl.ds(j * 16, 16)] = plsc.load_gather(ptv, [ordv])
            nsort[pl.ds(j * 16, 16)] = plsc.load_gather(ntv, [ordv])
            rsort[pl.ds(j * 16, 16)] = rsv

        rel_descs = [
            pltpu.async_copy(rsort.at[pl.ds(ch * _CH, _CH)],
                             out_r.at[pos2d.at[ch]], lsem)
            for ch in range(_PW // _CH)
        ]

        srcs = [hsort, psort, nsort]
        outs = [out_h, out_p, out_n]
        njobs = 3 * (_PW // _CH)

        def job(k):
            t, ch = k % 3, k // 3
            return srcs[t], outs[t], ch

        dg = {}
        dsc = {}
        for k in range(6):
            src, _, ch = job(k)
            dg[k] = pltpu.async_copy(table.at[src.at[pl.ds(ch * _CH, _CH)]],
                                     bufs[k], sems[k])
        for k in range(njobs):
            slot = k % 6
            dg[k].wait()
            _, out, ch = job(k)
            dsc[k] = pltpu.async_copy(bufs[slot], out.at[pos2d.at[ch]],
                                      sems[slot])
            if k + 6 < njobs:
                dsc[k].wait()
                src, _, ch2 = job(k + 6)
                dg[k + 6] = pltpu.async_copy(
                    table.at[src.at[pl.ds(ch2 * _CH, _CH)]],
                    bufs[slot], sems[slot])
        for k in range(njobs - 6, njobs):
            dsc[k].wait()
        for dsc_rel in rel_descs:
            dsc_rel.wait()

    return sg


_BK = 1024
_NB = _B // _BK


def _tc_body(rel_ref, hb_ref, pb_ref, nb_ref, rtab_ref, trans_ref, out_ref,
             s_ref, acc_ref):
    i = pl.program_id(0)
    rel = rel_ref[0, 0, :]
    rel3 = jnp.concatenate([rel, rel, rel], axis=0)
    lo = jnp.min(rel)
    hi = jnp.max(rel)

    s_ref[0:_BK, :] = hb_ref[...]
    s_ref[_BK:2 * _BK, :] = pb_ref[...]
    s_ref[2 * _BK:3 * _BK, :] = nb_ref[...]
    acc_ref[...] = jnp.zeros((3 * _BK, _DIM), jnp.float32)

    def body(r, carry):
        m = (rel3 == r).astype(jnp.float32)[:, None]
        w_r = trans_ref[r, :, :]
        acc_ref[...] += jnp.dot(s_ref[...] * m, w_r,
                                preferred_element_type=jnp.float32)
        return carry

    lax.fori_loop(lo, hi + 1, body, 0)

    th = acc_ref[0:_BK, :]
    tp = acc_ref[_BK:2 * _BK, :]
    tn = acc_ref[2 * _BK:3 * _BK, :]

    oh = (rel[:, None] == lax.broadcasted_iota(jnp.int32, (1, _RELATION_NUM), 1)
          ).astype(jnp.float32)
    remb = jnp.dot(oh, rtab_ref[...], preferred_element_type=jnp.float32)

    pos = jnp.sum(jnp.square(th + remb - tp), axis=1)
    neg = jnp.sum(jnp.square(th + remb - tn), axis=1)
    d = neg - pos
    ls = jnp.minimum(d, 0.0) - jnp.log1p(jnp.exp(-jnp.abs(d)))

    rows = lax.broadcasted_iota(jnp.int32, (8, _DIM), 0)
    cols = lax.broadcasted_iota(jnp.int32, (8, _DIM), 1)
    partial = (jnp.sum(ls) * (rows == 0) + jnp.sum(th * th) * (rows == 1)
               + jnp.sum(remb * remb) * (rows == 2)
               + jnp.sum(tp * tp) * (rows == 3)
               + jnp.sum(tn * tn) * (rows == 4)).astype(jnp.float32)

    @pl.when(i == 0)
    def _():
        out_ref[...] = jnp.zeros((8, _DIM), jnp.float32)

    out_ref[...] += partial

    @pl.when(i == _NB - 1)
    def _():
        vals = out_ref[...]
        inv = 1.0 / _DIM

        def tot(r):
            return jnp.sum(vals * (rows == r).astype(jnp.float32)) * inv

        kg = -tot(0) / _B
        l2 = (tot(1) + tot(2) + tot(3) + tot(4)) / (2.0 * _B)
        res = kg + _REG * l2
        out_ref[...] = res * ((rows == 0) & (cols == 0)).astype(jnp.float32)


def _tc_call(rows_h, rows_p, rows_n, rel3d, rtab, trans):
    return pl.pallas_call(
        _tc_body,
        grid=(_NB,),
        in_specs=[
            pl.BlockSpec((1, 1, _BK), lambda i: (i, 0, 0)),
            pl.BlockSpec((_BK, _DIM), lambda i: (i, 0)),
            pl.BlockSpec((_BK, _DIM), lambda i: (i, 0)),
            pl.BlockSpec((_BK, _DIM), lambda i: (i, 0)),
            pl.BlockSpec((_RELATION_NUM, _DIM), lambda i: (0, 0)),
            pl.BlockSpec((_RELATION_NUM, _DIM, _DIM), lambda i: (0, 0, 0)),
        ],
        out_specs=pl.BlockSpec((8, _DIM), lambda i: (0, 0)),
        out_shape=jax.ShapeDtypeStruct((8, _DIM), jnp.float32),
        scratch_shapes=[
            pltpu.VMEM((3 * _BK, _DIM), jnp.float32),
            pltpu.VMEM((3 * _BK, _DIM), jnp.float32),
        ],
    )(rel3d, rows_h, rows_p, rows_n, rtab, trans)


def kernel(user_entity_table, relation_table, trans_matrix, heads, relations,
           positive_tails, negative_tails):
    rows_h, rows_p, rows_n, rel_s = _make_sort_gather()(
        user_entity_table,
        heads.astype(jnp.int32),
        positive_tails.astype(jnp.int32),
        negative_tails.astype(jnp.int32),
        relations.astype(jnp.int32),
    )
    rel3d = rel_s.reshape(_NB, 1, _BK)
    out = _tc_call(rows_h, rows_p, rows_n, rel3d, relation_table, trans_matrix)
    return out[0, 0]
